# Initial kernel scaffold; baseline (speedup 1.0000x reference)
import jax
import jax.numpy as jnp
from jax.experimental import pallas as pl


def kernel(const_vals, W_const, b_const, emb_table, is_const, emb_type_idx):
    raise NotImplementedError("write your pallas kernel here")


# speedup vs baseline: 5.2796x; 5.2796x over previous
import functools

import jax
import jax.numpy as jnp
from jax.experimental import pallas as pl
from jax.experimental.pallas import tpu as pltpu

D_MODEL = 128
TOTAL_EMB = 150
N_TOKENS = 16384
_PAD_EMB = 160
_BLK = 2048
_NBLK = N_TOKENS // _BLK


def _tc_body(cv_ref, mask_ref, idx_ref, table_ref, w_ref, b_ref, out_ref):
    cv = cv_ref[0, 0, :]
    mask = mask_ref[0, 0, :]
    idx = idx_ref[0, 0, :]
    cols = jax.lax.broadcasted_iota(jnp.int32, (_BLK, _PAD_EMB), 1)
    onehot = (cols == idx[:, None]).astype(jnp.float32)
    emb_out = jax.lax.dot_general(
        onehot, table_ref[...],
        dimension_numbers=(((1,), (0,)), ((), ())),
        preferred_element_type=jnp.float32,
    )
    const_out = cv[:, None] * w_ref[0, :][None, :] + b_ref[0, :][None, :]
    out_ref[...] = jnp.where(mask[:, None] > 0.5, const_out, emb_out)


@jax.jit
def kernel(const_vals, W_const, b_const, emb_table, is_const, emb_type_idx):
    cv = const_vals.reshape(_NBLK, 1, _BLK)
    mask = is_const.astype(jnp.float32).reshape(_NBLK, 1, _BLK)
    idx = emb_type_idx.astype(jnp.int32).reshape(_NBLK, 1, _BLK)
    table = jnp.zeros((_PAD_EMB, D_MODEL), jnp.float32).at[:TOTAL_EMB].set(emb_table)
    w = W_const.reshape(1, D_MODEL)
    b = b_const.reshape(1, D_MODEL)

    grid = (_NBLK,)
    out = pl.pallas_call(
        _tc_body,
        grid=grid,
        in_specs=[
            pl.BlockSpec((1, 1, _BLK), lambda i: (i, 0, 0)),
            pl.BlockSpec((1, 1, _BLK), lambda i: (i, 0, 0)),
            pl.BlockSpec((1, 1, _BLK), lambda i: (i, 0, 0)),
            pl.BlockSpec((_PAD_EMB, D_MODEL), lambda i: (0, 0)),
            pl.BlockSpec((1, D_MODEL), lambda i: (0, 0)),
            pl.BlockSpec((1, D_MODEL), lambda i: (0, 0)),
        ],
        out_specs=pl.BlockSpec((_BLK, D_MODEL), lambda i: (i, 0)),
        out_shape=jax.ShapeDtypeStruct((N_TOKENS, D_MODEL), jnp.float32),
    )(cv, mask, idx, table, w, b)
    return out
